# Initial kernel scaffold; baseline (speedup 1.0000x reference)
import functools

import jax
import jax.numpy as jnp
from jax import lax
from jax.experimental import pallas as pl
from jax.experimental.pallas import tpu as pltpu
from jax.experimental.pallas import tpu_sc as plsc

_NC = 2
_NS = 16
_NW = _NC * _NS
_L = 16


def _sc_scores(u_pos, v_pos, v_neg_flat, U, V, *, B, D, NNEG, CH):
    BPW = B // _NW
    NCH = BPW // CH
    ND = NNEG * CH
    NV = D // _L

    mesh = plsc.VectorSubcoreMesh(
        core_axis_name="c", subcore_axis_name="s",
        num_cores=_NC, num_subcores=_NS)

    def body(u_hbm, v_hbm, n_hbm, U_hbm, V_hbm, sp_hbm, sn_hbm,
             idx_u, idx_v, idx_n, rows_u, rows_v, rows_n, sp_buf, sn_buf,
             sem):
        wid = lax.axis_index("s") * _NC + lax.axis_index("c")
        base = wid * BPW

        def chunk(c, carry):
            off = base + c * CH
            pltpu.sync_copy(u_hbm.at[pl.ds(off, CH)], idx_u)
            pltpu.sync_copy(v_hbm.at[pl.ds(off, CH)], idx_v)
            pltpu.sync_copy(n_hbm.at[pl.ds(off * NNEG, ND)], idx_n)
            cps = [pltpu.async_copy(U_hbm.at[idx_u], rows_u, sem),
                   pltpu.async_copy(V_hbm.at[idx_v], rows_v, sem)]
            for j in range(NNEG):
                cps.append(pltpu.async_copy(
                    V_hbm.at[idx_n.at[pl.ds(j * CH, CH)]],
                    rows_n.at[pl.ds(j * CH, CH)], sem))
            for cp in cps:
                cp.wait()

            def elem(e, carry2):
                u = [rows_u[e, pl.ds(_L * j, _L)] for j in range(NV)]
                v = [rows_v[e, pl.ds(_L * j, _L)] for j in range(NV)]
                p = u[0] * v[0]
                for j in range(1, NV):
                    p = p + u[j] * v[j]
                q = None
                for k in range(NNEG):
                    t = None
                    for j in range(NV):
                        r = rows_n[e * NNEG + k, pl.ds(_L * j, _L)]
                        t = u[j] * r if t is None else t + u[j] * r
                    q = t if q is None else q + t
                sp_buf[c * CH + e] = jnp.sum(p) * (1.0 / D)
                sn_buf[c * CH + e] = jnp.sum(q) * (1.0 / NNEG)
                return carry2

            lax.fori_loop(0, CH, elem, 0)
            return carry

        lax.fori_loop(0, NCH, chunk, 0)
        pltpu.sync_copy(sp_buf, sp_hbm.at[pl.ds(base, BPW)])
        pltpu.sync_copy(sn_buf, sn_hbm.at[pl.ds(base, BPW)])

    f = pl.kernel(
        body,
        out_type=[jax.ShapeDtypeStruct((B,), jnp.float32),
                  jax.ShapeDtypeStruct((B,), jnp.float32)],
        mesh=mesh,
        scratch_types=[
            pltpu.VMEM((CH,), jnp.int32),
            pltpu.VMEM((CH,), jnp.int32),
            pltpu.VMEM((ND,), jnp.int32),
            pltpu.VMEM((CH, D), jnp.float32),
            pltpu.VMEM((CH, D), jnp.float32),
            pltpu.VMEM((ND, D), jnp.float32),
            pltpu.VMEM((BPW,), jnp.float32),
            pltpu.VMEM((BPW,), jnp.float32),
            pltpu.SemaphoreType.DMA,
        ],
    )
    return f(u_pos, v_pos, v_neg_flat, U, V)


def _loss_body(sp_ref, sn_ref, o_ref, *, B):
    sp = sp_ref[...]
    sn = sn_ref[...]
    lt = jnp.minimum(sp, 0.0) - jnp.log1p(jnp.exp(-jnp.abs(sp)))
    ls = jnp.minimum(-sn, 0.0) - jnp.log1p(jnp.exp(-jnp.abs(sn)))
    o_ref[0, 0] = -(jnp.sum(lt) + jnp.sum(ls)) / B


def kernel(u_pos, v_pos, v_neg, batch_size, U, V):
    B = u_pos.shape[0]
    D = U.shape[1]
    NNEG = v_neg.shape[1]
    sp, sn = _sc_scores(u_pos, v_pos, v_neg.reshape(-1), U, V,
                        B=B, D=D, NNEG=NNEG, CH=128)
    loss = pl.pallas_call(
        functools.partial(_loss_body, B=B),
        out_shape=jax.ShapeDtypeStruct((1, 1), jnp.float32),
        in_specs=[pl.BlockSpec(memory_space=pltpu.VMEM),
                  pl.BlockSpec(memory_space=pltpu.VMEM)],
        out_specs=pl.BlockSpec(memory_space=pltpu.SMEM),
    )(sp.reshape(128, -1), sn.reshape(128, -1))
    return loss[0, 0]


# speedup vs baseline: 1.7334x; 1.7334x over previous
import functools

import jax
import jax.numpy as jnp
from jax import lax
from jax.experimental import pallas as pl
from jax.experimental.pallas import tpu as pltpu
from jax.experimental.pallas import tpu_sc as plsc

_NC = 2
_NS = 16
_NW = _NC * _NS
_L = 16


def _sc_scores(u_pos, v_pos, v_neg_flat, U, V, *, B, D, NNEG, CH):
    BPW = B // _NW
    NCH = BPW // CH
    ND = NNEG * CH
    NV = D // _L

    mesh = plsc.VectorSubcoreMesh(
        core_axis_name="c", subcore_axis_name="s",
        num_cores=_NC, num_subcores=_NS)

    def body(u_hbm, v_hbm, n_hbm, U_hbm, V_hbm, sp_hbm, sn_hbm,
             idx_u, idx_v, idx_n, rows_u, rows_v, rows_n, sp_buf, sn_buf,
             tr_p, tr_q, sem):
        wid = lax.axis_index("s") * _NC + lax.axis_index("c")
        base = wid * BPW
        colbase = lax.iota(jnp.int32, 16) * 16

        def chunk(c, carry):
            off = base + c * CH
            pltpu.sync_copy(u_hbm.at[pl.ds(off, CH)], idx_u)
            pltpu.sync_copy(v_hbm.at[pl.ds(off, CH)], idx_v)
            pltpu.sync_copy(n_hbm.at[pl.ds(off * NNEG, ND)], idx_n)
            cps = [pltpu.async_copy(U_hbm.at[idx_u], rows_u, sem),
                   pltpu.async_copy(V_hbm.at[idx_v], rows_v, sem)]
            for j in range(NNEG):
                cps.append(pltpu.async_copy(
                    V_hbm.at[idx_n.at[pl.ds(j * CH, CH)]],
                    rows_n.at[pl.ds(j * CH, CH)], sem))
            for cp in cps:
                cp.wait()

            def group(g, carry2):
                eb = g * _L
                for i in range(_L):
                    e = eb + i
                    u = [rows_u[e, pl.ds(_L * j, _L)] for j in range(NV)]
                    v = [rows_v[e, pl.ds(_L * j, _L)] for j in range(NV)]
                    p = u[0] * v[0]
                    for j in range(1, NV):
                        p = p + u[j] * v[j]
                    q = None
                    for k in range(NNEG):
                        t = None
                        for j in range(NV):
                            r = rows_n[e * NNEG + k, pl.ds(_L * j, _L)]
                            t = u[j] * r if t is None else t + u[j] * r
                        q = t if q is None else q + t
                    tr_p[pl.ds(i * _L, _L)] = p
                    tr_q[pl.ds(i * _L, _L)] = q
                sp = None
                sn = None
                for j in range(_L):
                    tp = plsc.load_gather(tr_p, [colbase + j])
                    tq = plsc.load_gather(tr_q, [colbase + j])
                    sp = tp if sp is None else sp + tp
                    sn = tq if sn is None else sn + tq
                sp_buf[pl.ds(c * CH + eb, _L)] = sp * (1.0 / D)
                sn_buf[pl.ds(c * CH + eb, _L)] = sn * (1.0 / NNEG)
                return carry2

            lax.fori_loop(0, CH // _L, group, 0)
            return carry

        lax.fori_loop(0, NCH, chunk, 0)
        pltpu.sync_copy(sp_buf, sp_hbm.at[pl.ds(base, BPW)])
        pltpu.sync_copy(sn_buf, sn_hbm.at[pl.ds(base, BPW)])

    f = pl.kernel(
        body,
        out_type=[jax.ShapeDtypeStruct((B,), jnp.float32),
                  jax.ShapeDtypeStruct((B,), jnp.float32)],
        mesh=mesh,
        compiler_params=pltpu.CompilerParams(needs_layout_passes=False,
                                             use_tc_tiling_on_sc=False),
        scratch_types=[
            pltpu.VMEM((CH,), jnp.int32),
            pltpu.VMEM((CH,), jnp.int32),
            pltpu.VMEM((ND,), jnp.int32),
            pltpu.VMEM((CH, D), jnp.float32),
            pltpu.VMEM((CH, D), jnp.float32),
            pltpu.VMEM((ND, D), jnp.float32),
            pltpu.VMEM((BPW,), jnp.float32),
            pltpu.VMEM((BPW,), jnp.float32),
            pltpu.VMEM((_L * _L,), jnp.float32),
            pltpu.VMEM((_L * _L,), jnp.float32),
            pltpu.SemaphoreType.DMA,
        ],
    )
    return f(u_pos, v_pos, v_neg_flat, U, V)


def _loss_body(sp_ref, sn_ref, o_ref, *, B):
    sp = sp_ref[...]
    sn = sn_ref[...]
    lt = jnp.minimum(sp, 0.0) - jnp.log1p(jnp.exp(-jnp.abs(sp)))
    ls = jnp.minimum(-sn, 0.0) - jnp.log1p(jnp.exp(-jnp.abs(sn)))
    o_ref[0, 0] = -(jnp.sum(lt) + jnp.sum(ls)) / B


def kernel(u_pos, v_pos, v_neg, batch_size, U, V):
    B = u_pos.shape[0]
    D = U.shape[1]
    NNEG = v_neg.shape[1]
    sp, sn = _sc_scores(u_pos, v_pos, v_neg.reshape(-1), U, V,
                        B=B, D=D, NNEG=NNEG, CH=128)
    loss = pl.pallas_call(
        functools.partial(_loss_body, B=B),
        out_shape=jax.ShapeDtypeStruct((1, 1), jnp.float32),
        in_specs=[pl.BlockSpec(memory_space=pltpu.VMEM),
                  pl.BlockSpec(memory_space=pltpu.VMEM)],
        out_specs=pl.BlockSpec(memory_space=pltpu.SMEM),
    )(sp.reshape(128, -1), sn.reshape(128, -1))
    return loss[0, 0]
